# baseline (speedup 1.0000x reference)
import functools

import jax
import jax.numpy as jnp
from jax import lax
from jax.experimental import pallas as pl
from jax.experimental.pallas import tpu as pltpu
from jax.experimental.pallas import tpu_sc as plsc

N_NODES = 10000
N_EDGES = 320000
D_IN = 128
D_HID = 256
D_OUT = 128

NC = 2
NS = 16
NW = NC * NS
CHUNK = 80
NSTEPS = 128
E_PAD = NW * NSTEPS * CHUNK
NBUF = 1
NGROUPS = NSTEPS // NBUF
N_PAD = 10240
ROWS_PER_TILE = N_PAD // NS
ZROWS = CHUNK


def _seg_sum_body(with_deg, x_hbm, src_hbm, dst_hbm, *refs):
    if with_deg:
        (acc_out, deg_out, s_a, t_a, r_a,
         ones_v, zdeg_v, acc_sh, deg_sh, sem_i, sem_g, sem_s, sem_d) = refs
    else:
        (acc_out, s_a, t_a, r_a, acc_sh, sem_i, sem_g, sem_s) = refs
        deg_out = deg_sh = ones_v = zdeg_v = sem_d = None

    c = lax.axis_index("c")
    s = lax.axis_index("s")
    wid = s * NC + c

    def zb(i, _):
        r = i // 8
        col = (i % 8) * 16
        r_a[r, pl.ds(col, 16)] = jnp.zeros((16,), jnp.float32)
        return 0
    lax.fori_loop(0, ZROWS * 8, zb, 0)
    if with_deg:
        def ob(i, _):
            ones_v[pl.ds(i * 16, 16)] = jnp.ones((16,), jnp.float32)
            return 0
        lax.fori_loop(0, CHUNK // 16, ob, 0)

        def zd(i, _):
            zdeg_v[pl.ds(i * 16, 16)] = jnp.zeros((16,), jnp.float32)
            return 0
        lax.fori_loop(0, ROWS_PER_TILE // 16, zd, 0)

    r0 = s * ROWS_PER_TILE
    for k in range(ROWS_PER_TILE // ZROWS):
        pltpu.sync_copy(r_a, acc_sh.at[pl.ds(r0 + k * ZROWS, ZROWS)])
    if with_deg:
        pltpu.sync_copy(zdeg_v, deg_sh.at[pl.ds(r0, ROWS_PER_TILE)])

    plsc.subcore_barrier()

    def step(i, _):
        j = pl.multiple_of((wid * NSTEPS + i) * CHUNK, CHUNK)
        pltpu.sync_copy(src_hbm.at[pl.ds(j, CHUNK)], s_a)
        pltpu.sync_copy(dst_hbm.at[pl.ds(j, CHUNK)], t_a)
        pltpu.async_copy(x_hbm.at[s_a], r_a, sem_g).wait()
        pltpu.sync_copy(r_a, acc_sh.at[t_a], add=True)
        if with_deg:
            pltpu.sync_copy(ones_v, deg_sh.at[t_a], add=True)
        return 0
    lax.fori_loop(0, NSTEPS, step, 0)
    plsc.subcore_barrier()

    pltpu.sync_copy(acc_sh.at[pl.ds(r0, ROWS_PER_TILE)],
                    acc_out.at[c, pl.ds(r0, ROWS_PER_TILE)])
    if with_deg:
        pltpu.sync_copy(deg_sh.at[pl.ds(r0, ROWS_PER_TILE)],
                        deg_out.at[pl.ds(c * N_PAD + r0, ROWS_PER_TILE)])


def _make_seg_sum(with_deg, d):
    srcb = [pltpu.VMEM((CHUNK,), jnp.int32) for _ in range(NBUF)]
    dstb = [pltpu.VMEM((CHUNK,), jnp.int32) for _ in range(NBUF)]
    rows = [pltpu.VMEM((CHUNK, d), jnp.float32) for _ in range(NBUF)]
    if with_deg:
        out_type = (jax.ShapeDtypeStruct((NC, N_PAD, d), jnp.float32),
                    jax.ShapeDtypeStruct((NC * N_PAD,), jnp.float32))
        scratch = srcb + dstb + rows + [
            pltpu.VMEM((CHUNK,), jnp.float32),
            pltpu.VMEM((ROWS_PER_TILE,), jnp.float32),
            pltpu.VMEM_SHARED((N_PAD, d), jnp.float32),
            pltpu.VMEM_SHARED((N_PAD,), jnp.float32),
            pltpu.SemaphoreType.DMA,
            pltpu.SemaphoreType.DMA,
            pltpu.SemaphoreType.DMA,
            pltpu.SemaphoreType.DMA,
        ]
    else:
        out_type = jax.ShapeDtypeStruct((NC, N_PAD, d), jnp.float32)
        scratch = srcb + dstb + rows + [
            pltpu.VMEM_SHARED((N_PAD, d), jnp.float32),
            pltpu.SemaphoreType.DMA,
            pltpu.SemaphoreType.DMA,
            pltpu.SemaphoreType.DMA,
        ]
    return pl.kernel(
        functools.partial(_seg_sum_body, with_deg),
        out_type=out_type,
        mesh=plsc.VectorSubcoreMesh(core_axis_name="c", subcore_axis_name="s"),
        scratch_types=scratch,
    )


_BLK = 1024
_GRID = N_PAD // _BLK


def _dot(a, b):
    return jnp.dot(a, b, preferred_element_type=jnp.float32,
                   precision=lax.Precision.HIGHEST)


def _mid_body(x_ref, a0_ref, a1_ref, d0_ref, d1_ref, ws1_ref, wn1_ref,
              b1_ref, ws2_ref, wn2_ref, b2_ref, u_ref, p2_ref):
    deg = jnp.maximum(d0_ref[...] + d1_ref[...], 1.0)
    mean = (a0_ref[...] + a1_ref[...]) / deg
    h = _dot(x_ref[...], ws1_ref[...]) + _dot(mean, wn1_ref[...]) + b1_ref[...]
    h = jnp.maximum(h, 0.0)
    u_ref[...] = _dot(h, ws2_ref[...]) + b2_ref[...]
    p2_ref[...] = _dot(h, wn2_ref[...])


def _fin_body(u_ref, a0_ref, a1_ref, d0_ref, d1_ref, out_ref):
    deg = jnp.maximum(d0_ref[...] + d1_ref[...], 1.0)
    out_ref[...] = u_ref[...] + (a0_ref[...] + a1_ref[...]) / deg


def _row_spec(d):
    return pl.BlockSpec((_BLK, d), lambda i: (i, 0))


def _full_spec(r, c):
    return pl.BlockSpec((r, c), lambda i: (0, 0))


def kernel(x, edge_index, W_self1, W_neigh1, b1, W_self2, W_neigh2, b2):
    src = edge_index[0].astype(jnp.int32)
    dst = edge_index[1].astype(jnp.int32)
    src2 = jnp.pad(src, (0, E_PAD - N_EDGES))
    pad_dst = N_NODES + jnp.arange(E_PAD - N_EDGES, dtype=jnp.int32) % (N_PAD - N_NODES)
    dst2 = jnp.concatenate([dst, pad_dst])
    x_pad = jnp.pad(x, ((0, N_PAD - N_NODES), (0, 0)))

    seg1 = _make_seg_sum(True, D_IN)
    agg1, deg_flat = seg1(x_pad, src2, dst2)
    deg2 = deg_flat.reshape(NC, N_PAD)
    d0 = deg2[0][:, None]
    d1 = deg2[1][:, None]

    mid = pl.pallas_call(
        _mid_body,
        grid=(_GRID,),
        in_specs=[
            _row_spec(D_IN), _row_spec(D_IN), _row_spec(D_IN),
            _row_spec(1), _row_spec(1),
            _full_spec(D_IN, D_HID), _full_spec(D_IN, D_HID),
            _full_spec(1, D_HID),
            _full_spec(D_HID, D_OUT), _full_spec(D_HID, D_OUT),
            _full_spec(1, D_OUT),
        ],
        out_specs=[_row_spec(D_OUT), _row_spec(D_OUT)],
        out_shape=[
            jax.ShapeDtypeStruct((N_PAD, D_OUT), jnp.float32),
            jax.ShapeDtypeStruct((N_PAD, D_OUT), jnp.float32),
        ],
    )
    u, p2 = mid(x_pad, agg1[0], agg1[1], d0, d1,
                W_self1, W_neigh1, b1.reshape(1, D_HID),
                W_self2, W_neigh2, b2.reshape(1, D_OUT))

    seg2 = _make_seg_sum(False, D_OUT)
    agg2 = seg2(p2, src2, dst2)

    fin = pl.pallas_call(
        _fin_body,
        grid=(_GRID,),
        in_specs=[
            _row_spec(D_OUT), _row_spec(D_OUT), _row_spec(D_OUT),
            _row_spec(1), _row_spec(1),
        ],
        out_specs=_row_spec(D_OUT),
        out_shape=jax.ShapeDtypeStruct((N_PAD, D_OUT), jnp.float32),
    )
    out = fin(u, agg2[0], agg2[1], d0, d1)
    return out[:N_NODES]


# speedup vs baseline: 1.9261x; 1.9261x over previous
import functools

import jax
import jax.numpy as jnp
from jax import lax
from jax.experimental import pallas as pl
from jax.experimental.pallas import tpu as pltpu
from jax.experimental.pallas import tpu_sc as plsc

N_NODES = 10000
N_EDGES = 320000
D_IN = 128
D_HID = 256
D_OUT = 128

NC = 2
NS = 16
NW = NC * NS
CHUNK = 80
NSTEPS = 128
E_PAD = NW * NSTEPS * CHUNK
NBUF = 1
NGROUPS = NSTEPS // NBUF
N_PAD = 10240
ROWS_PER_TILE = N_PAD // NS
ZROWS = CHUNK


def _seg_sum_body(with_deg, x_hbm, src_hbm, dst_hbm, *refs):
    if with_deg:
        (acc_out, deg_out, s_a, t_a, r_a,
         ones_v, zdeg_v, acc_sh, deg_sh, sem_i, sem_g, sem_s, sem_d) = refs
    else:
        (acc_out, s_a, t_a, r_a, acc_sh, sem_i, sem_g, sem_s) = refs
        deg_out = deg_sh = ones_v = zdeg_v = sem_d = None

    c = lax.axis_index("c")
    s = lax.axis_index("s")
    wid = s * NC + c

    def zb(i, _):
        r = i // 8
        col = (i % 8) * 16
        r_a[r, pl.ds(col, 16)] = jnp.zeros((16,), jnp.float32)
        return 0
    lax.fori_loop(0, ZROWS * 8, zb, 0)
    if with_deg:
        def ob(i, _):
            ones_v[pl.ds(i * 16, 16)] = jnp.ones((16,), jnp.float32)
            return 0
        lax.fori_loop(0, CHUNK // 16, ob, 0)

        def zd(i, _):
            zdeg_v[pl.ds(i * 16, 16)] = jnp.zeros((16,), jnp.float32)
            return 0
        lax.fori_loop(0, ROWS_PER_TILE // 16, zd, 0)

    r0 = s * ROWS_PER_TILE
    for k in range(ROWS_PER_TILE // ZROWS):
        pltpu.sync_copy(r_a, acc_sh.at[pl.ds(r0 + k * ZROWS, ZROWS)])
    if with_deg:
        pltpu.sync_copy(zdeg_v, deg_sh.at[pl.ds(r0, ROWS_PER_TILE)])

    plsc.subcore_barrier()

    def step(i, _):
        j = pl.multiple_of((wid * NSTEPS + i) * CHUNK, CHUNK)
        pltpu.sync_copy(src_hbm.at[pl.ds(j, CHUNK)], s_a)
        pltpu.sync_copy(dst_hbm.at[pl.ds(j, CHUNK)], t_a)
        pltpu.async_copy(x_hbm.at[s_a], r_a, sem_g).wait()
        pltpu.sync_copy(r_a, acc_sh.at[t_a], add=True)
        if with_deg:
            pltpu.sync_copy(ones_v, deg_sh.at[t_a], add=True)
        return 0
    lax.fori_loop(0, NSTEPS, step, 0)
    plsc.subcore_barrier()

    pltpu.sync_copy(acc_sh.at[pl.ds(r0, ROWS_PER_TILE)],
                    acc_out.at[c, pl.ds(r0, ROWS_PER_TILE)])
    if with_deg:
        pltpu.sync_copy(deg_sh.at[pl.ds(r0, ROWS_PER_TILE)],
                        deg_out.at[pl.ds(c * N_PAD + r0, ROWS_PER_TILE)])


def _make_seg_sum(with_deg, d):
    srcb = [pltpu.VMEM((CHUNK,), jnp.int32) for _ in range(NBUF)]
    dstb = [pltpu.VMEM((CHUNK,), jnp.int32) for _ in range(NBUF)]
    rows = [pltpu.VMEM((CHUNK, d), jnp.float32) for _ in range(NBUF)]
    if with_deg:
        out_type = (jax.ShapeDtypeStruct((NC, N_PAD, d), jnp.float32),
                    jax.ShapeDtypeStruct((NC * N_PAD,), jnp.float32))
        scratch = srcb + dstb + rows + [
            pltpu.VMEM((CHUNK,), jnp.float32),
            pltpu.VMEM((ROWS_PER_TILE,), jnp.float32),
            pltpu.VMEM_SHARED((N_PAD, d), jnp.float32),
            pltpu.VMEM_SHARED((N_PAD,), jnp.float32),
            pltpu.SemaphoreType.DMA,
            pltpu.SemaphoreType.DMA,
            pltpu.SemaphoreType.DMA,
            pltpu.SemaphoreType.DMA,
        ]
    else:
        out_type = jax.ShapeDtypeStruct((NC, N_PAD, d), jnp.float32)
        scratch = srcb + dstb + rows + [
            pltpu.VMEM_SHARED((N_PAD, d), jnp.float32),
            pltpu.SemaphoreType.DMA,
            pltpu.SemaphoreType.DMA,
            pltpu.SemaphoreType.DMA,
        ]
    return pl.kernel(
        functools.partial(_seg_sum_body, with_deg),
        out_type=out_type,
        mesh=plsc.VectorSubcoreMesh(core_axis_name="c", subcore_axis_name="s"),
        scratch_types=scratch,
    )


_BLK = 1024
_GRID = N_PAD // _BLK


def _dot(a, b):
    return jnp.dot(a, b, preferred_element_type=jnp.float32,
                   precision=lax.Precision.HIGHEST)


def _mid_body(x_ref, a0_ref, a1_ref, d0_ref, d1_ref, ws1_ref, wn1_ref,
              b1_ref, ws2_ref, wn2_ref, b2_ref, u_ref, p2_ref):
    deg = jnp.maximum(d0_ref[...] + d1_ref[...], 1.0)
    mean = (a0_ref[...] + a1_ref[...]) / deg
    h = _dot(x_ref[...], ws1_ref[...]) + _dot(mean, wn1_ref[...]) + b1_ref[...]
    h = jnp.maximum(h, 0.0)
    u_ref[...] = _dot(h, ws2_ref[...]) + b2_ref[...]
    p2_ref[...] = _dot(h, wn2_ref[...])


def _fin_body(u_ref, a0_ref, a1_ref, d0_ref, d1_ref, out_ref):
    deg = jnp.maximum(d0_ref[...] + d1_ref[...], 1.0)
    out_ref[...] = u_ref[...] + (a0_ref[...] + a1_ref[...]) / deg


def _row_spec(d):
    return pl.BlockSpec((_BLK, d), lambda i: (i, 0))


def _full_spec(r, c):
    return pl.BlockSpec((r, c), lambda i: (0, 0))


def kernel(x, edge_index, W_self1, W_neigh1, b1, W_self2, W_neigh2, b2):
    src = edge_index[0].astype(jnp.int32)
    dst = edge_index[1].astype(jnp.int32)
    npad_e = E_PAD - N_EDGES
    pad_src = jnp.arange(npad_e, dtype=jnp.int32) % N_NODES
    pad_dst = N_NODES + jnp.arange(npad_e, dtype=jnp.int32) % (N_PAD - N_NODES)
    src2 = jnp.concatenate([src, pad_src])
    dst2 = jnp.concatenate([dst, pad_dst])
    x_pad = jnp.pad(x, ((0, N_PAD - N_NODES), (0, 0)))

    seg1 = _make_seg_sum(True, D_IN)
    agg1, deg_flat = seg1(x_pad, src2, dst2)
    deg2 = deg_flat.reshape(NC, N_PAD)
    d0 = deg2[0][:, None]
    d1 = deg2[1][:, None]

    mid = pl.pallas_call(
        _mid_body,
        grid=(_GRID,),
        in_specs=[
            _row_spec(D_IN), _row_spec(D_IN), _row_spec(D_IN),
            _row_spec(1), _row_spec(1),
            _full_spec(D_IN, D_HID), _full_spec(D_IN, D_HID),
            _full_spec(1, D_HID),
            _full_spec(D_HID, D_OUT), _full_spec(D_HID, D_OUT),
            _full_spec(1, D_OUT),
        ],
        out_specs=[_row_spec(D_OUT), _row_spec(D_OUT)],
        out_shape=[
            jax.ShapeDtypeStruct((N_PAD, D_OUT), jnp.float32),
            jax.ShapeDtypeStruct((N_PAD, D_OUT), jnp.float32),
        ],
    )
    u, p2 = mid(x_pad, agg1[0], agg1[1], d0, d1,
                W_self1, W_neigh1, b1.reshape(1, D_HID),
                W_self2, W_neigh2, b2.reshape(1, D_OUT))

    seg2 = _make_seg_sum(False, D_OUT)
    agg2 = seg2(p2, src2, dst2)

    fin = pl.pallas_call(
        _fin_body,
        grid=(_GRID,),
        in_specs=[
            _row_spec(D_OUT), _row_spec(D_OUT), _row_spec(D_OUT),
            _row_spec(1), _row_spec(1),
        ],
        out_specs=_row_spec(D_OUT),
        out_shape=jax.ShapeDtypeStruct((N_PAD, D_OUT), jnp.float32),
    )
    out = fin(u, agg2[0], agg2[1], d0, d1)
    return out[:N_NODES]


# speedup vs baseline: 2.3266x; 1.2079x over previous
import functools

import jax
import jax.numpy as jnp
from jax import lax
from jax.experimental import pallas as pl
from jax.experimental.pallas import tpu as pltpu
from jax.experimental.pallas import tpu_sc as plsc

N_NODES = 10000
N_EDGES = 320000
D_IN = 128
D_HID = 256
D_OUT = 128

NC = 2
NS = 16
NW = NC * NS
CHUNK = 128
NSTEPS = 80
E_PAD = NW * NSTEPS * CHUNK
NBUF = 1
NGROUPS = NSTEPS // NBUF
N_PAD = 10240
ROWS_PER_TILE = N_PAD // NS
ZROWS = CHUNK


def _seg_sum_body(with_deg, x_hbm, src_hbm, dst_hbm, *refs):
    if with_deg:
        (acc_out, deg_out, s_a, t_a, r_a,
         ones_v, zdeg_v, acc_sh, deg_sh, sem_i, sem_g, sem_s, sem_d) = refs
    else:
        (acc_out, s_a, t_a, r_a, acc_sh, sem_i, sem_g, sem_s) = refs
        deg_out = deg_sh = ones_v = zdeg_v = sem_d = None

    c = lax.axis_index("c")
    s = lax.axis_index("s")
    wid = s * NC + c

    def zb(i, _):
        r = i // 8
        col = (i % 8) * 16
        r_a[r, pl.ds(col, 16)] = jnp.zeros((16,), jnp.float32)
        return 0
    lax.fori_loop(0, ZROWS * 8, zb, 0)
    if with_deg:
        def ob(i, _):
            ones_v[pl.ds(i * 16, 16)] = jnp.ones((16,), jnp.float32)
            return 0
        lax.fori_loop(0, CHUNK // 16, ob, 0)

        def zd(i, _):
            zdeg_v[pl.ds(i * 16, 16)] = jnp.zeros((16,), jnp.float32)
            return 0
        lax.fori_loop(0, ROWS_PER_TILE // 16, zd, 0)

    r0 = s * ROWS_PER_TILE
    for k in range(ROWS_PER_TILE // ZROWS):
        pltpu.sync_copy(r_a, acc_sh.at[pl.ds(r0 + k * ZROWS, ZROWS)])
    if with_deg:
        pltpu.sync_copy(zdeg_v, deg_sh.at[pl.ds(r0, ROWS_PER_TILE)])

    plsc.subcore_barrier()

    def step(i, _):
        j = pl.multiple_of((wid * NSTEPS + i) * CHUNK, CHUNK)
        pltpu.sync_copy(src_hbm.at[pl.ds(j, CHUNK)], s_a)
        pltpu.sync_copy(dst_hbm.at[pl.ds(j, CHUNK)], t_a)
        pltpu.async_copy(x_hbm.at[s_a], r_a, sem_g).wait()
        pltpu.sync_copy(r_a, acc_sh.at[t_a], add=True)
        if with_deg:
            pltpu.sync_copy(ones_v, deg_sh.at[t_a], add=True)
        return 0
    lax.fori_loop(0, NSTEPS, step, 0)
    plsc.subcore_barrier()

    pltpu.sync_copy(acc_sh.at[pl.ds(r0, ROWS_PER_TILE)],
                    acc_out.at[c, pl.ds(r0, ROWS_PER_TILE)])
    if with_deg:
        pltpu.sync_copy(deg_sh.at[pl.ds(r0, ROWS_PER_TILE)],
                        deg_out.at[pl.ds(c * N_PAD + r0, ROWS_PER_TILE)])


def _make_seg_sum(with_deg, d):
    srcb = [pltpu.VMEM((CHUNK,), jnp.int32) for _ in range(NBUF)]
    dstb = [pltpu.VMEM((CHUNK,), jnp.int32) for _ in range(NBUF)]
    rows = [pltpu.VMEM((CHUNK, d), jnp.float32) for _ in range(NBUF)]
    if with_deg:
        out_type = (jax.ShapeDtypeStruct((NC, N_PAD, d), jnp.float32),
                    jax.ShapeDtypeStruct((NC * N_PAD,), jnp.float32))
        scratch = srcb + dstb + rows + [
            pltpu.VMEM((CHUNK,), jnp.float32),
            pltpu.VMEM((ROWS_PER_TILE,), jnp.float32),
            pltpu.VMEM_SHARED((N_PAD, d), jnp.float32),
            pltpu.VMEM_SHARED((N_PAD,), jnp.float32),
            pltpu.SemaphoreType.DMA,
            pltpu.SemaphoreType.DMA,
            pltpu.SemaphoreType.DMA,
            pltpu.SemaphoreType.DMA,
        ]
    else:
        out_type = jax.ShapeDtypeStruct((NC, N_PAD, d), jnp.float32)
        scratch = srcb + dstb + rows + [
            pltpu.VMEM_SHARED((N_PAD, d), jnp.float32),
            pltpu.SemaphoreType.DMA,
            pltpu.SemaphoreType.DMA,
            pltpu.SemaphoreType.DMA,
        ]
    return pl.kernel(
        functools.partial(_seg_sum_body, with_deg),
        out_type=out_type,
        mesh=plsc.VectorSubcoreMesh(core_axis_name="c", subcore_axis_name="s"),
        scratch_types=scratch,
    )


_BLK = 1024
_GRID = N_PAD // _BLK


def _dot(a, b):
    return jnp.dot(a, b, preferred_element_type=jnp.float32,
                   precision=lax.Precision.HIGHEST)


def _mid_body(x_ref, a0_ref, a1_ref, d0_ref, d1_ref, ws1_ref, wn1_ref,
              b1_ref, ws2_ref, wn2_ref, b2_ref, u_ref, p2_ref):
    deg = jnp.maximum(d0_ref[...] + d1_ref[...], 1.0)
    mean = (a0_ref[...] + a1_ref[...]) / deg
    h = _dot(x_ref[...], ws1_ref[...]) + _dot(mean, wn1_ref[...]) + b1_ref[...]
    h = jnp.maximum(h, 0.0)
    u_ref[...] = _dot(h, ws2_ref[...]) + b2_ref[...]
    p2_ref[...] = _dot(h, wn2_ref[...])


def _fin_body(u_ref, a0_ref, a1_ref, d0_ref, d1_ref, out_ref):
    deg = jnp.maximum(d0_ref[...] + d1_ref[...], 1.0)
    out_ref[...] = u_ref[...] + (a0_ref[...] + a1_ref[...]) / deg


def _row_spec(d):
    return pl.BlockSpec((_BLK, d), lambda i: (i, 0))


def _full_spec(r, c):
    return pl.BlockSpec((r, c), lambda i: (0, 0))


def kernel(x, edge_index, W_self1, W_neigh1, b1, W_self2, W_neigh2, b2):
    src = edge_index[0].astype(jnp.int32)
    dst = edge_index[1].astype(jnp.int32)
    npad_e = E_PAD - N_EDGES
    pad_src = jnp.arange(npad_e, dtype=jnp.int32) % N_NODES
    pad_dst = N_NODES + jnp.arange(npad_e, dtype=jnp.int32) % (N_PAD - N_NODES)
    src2 = jnp.concatenate([src, pad_src])
    dst2 = jnp.concatenate([dst, pad_dst])
    x_pad = jnp.pad(x, ((0, N_PAD - N_NODES), (0, 0)))

    seg1 = _make_seg_sum(True, D_IN)
    agg1, deg_flat = seg1(x_pad, src2, dst2)
    deg2 = deg_flat.reshape(NC, N_PAD)
    d0 = deg2[0][:, None]
    d1 = deg2[1][:, None]

    mid = pl.pallas_call(
        _mid_body,
        grid=(_GRID,),
        in_specs=[
            _row_spec(D_IN), _row_spec(D_IN), _row_spec(D_IN),
            _row_spec(1), _row_spec(1),
            _full_spec(D_IN, D_HID), _full_spec(D_IN, D_HID),
            _full_spec(1, D_HID),
            _full_spec(D_HID, D_OUT), _full_spec(D_HID, D_OUT),
            _full_spec(1, D_OUT),
        ],
        out_specs=[_row_spec(D_OUT), _row_spec(D_OUT)],
        out_shape=[
            jax.ShapeDtypeStruct((N_PAD, D_OUT), jnp.float32),
            jax.ShapeDtypeStruct((N_PAD, D_OUT), jnp.float32),
        ],
    )
    u, p2 = mid(x_pad, agg1[0], agg1[1], d0, d1,
                W_self1, W_neigh1, b1.reshape(1, D_HID),
                W_self2, W_neigh2, b2.reshape(1, D_OUT))

    seg2 = _make_seg_sum(False, D_OUT)
    agg2 = seg2(p2, src2, dst2)

    fin = pl.pallas_call(
        _fin_body,
        grid=(_GRID,),
        in_specs=[
            _row_spec(D_OUT), _row_spec(D_OUT), _row_spec(D_OUT),
            _row_spec(1), _row_spec(1),
        ],
        out_specs=_row_spec(D_OUT),
        out_shape=jax.ShapeDtypeStruct((N_PAD, D_OUT), jnp.float32),
    )
    out = fin(u, agg2[0], agg2[1], d0, d1)
    return out[:N_NODES]


# speedup vs baseline: 3.3455x; 1.4379x over previous
import functools

import jax
import jax.numpy as jnp
from jax import lax
from jax.experimental import pallas as pl
from jax.experimental.pallas import tpu as pltpu
from jax.experimental.pallas import tpu_sc as plsc

N_NODES = 10000
N_EDGES = 320000
D_IN = 128
D_HID = 256
D_OUT = 128

NC = 2
NS = 16
NW = NC * NS
CHUNK = 128
NSTEPS = 80
E_PAD = NW * NSTEPS * CHUNK
NBUF = 2
E_ALLOC = E_PAD + CHUNK
N_PAD = 10240
ROWS_PER_TILE = N_PAD // NS
ZROWS = CHUNK


def _seg_sum_body(with_deg, x_hbm, src_hbm, dst_hbm, *refs):
    if with_deg:
        (acc_out, deg_out, s_a, s_b, t_a, t_b, r_a, r_b,
         ones_v, zdeg_v, acc_sh, deg_sh, sem_g) = refs
    else:
        (acc_out, s_a, s_b, t_a, t_b, r_a, r_b, acc_sh, sem_g) = refs
        deg_out = deg_sh = ones_v = zdeg_v = None

    c = lax.axis_index("c")
    s = lax.axis_index("s")
    wid = s * NC + c

    def zb(i, _):
        r = i // 8
        col = (i % 8) * 16
        r_a[r, pl.ds(col, 16)] = jnp.zeros((16,), jnp.float32)
        return 0
    lax.fori_loop(0, ZROWS * 8, zb, 0)
    if with_deg:
        def ob(i, _):
            ones_v[pl.ds(i * 16, 16)] = jnp.ones((16,), jnp.float32)
            return 0
        lax.fori_loop(0, CHUNK // 16, ob, 0)

        def zd(i, _):
            zdeg_v[pl.ds(i * 16, 16)] = jnp.zeros((16,), jnp.float32)
            return 0
        lax.fori_loop(0, ROWS_PER_TILE // 16, zd, 0)

    r0 = s * ROWS_PER_TILE
    for k in range(ROWS_PER_TILE // ZROWS):
        pltpu.sync_copy(r_a, acc_sh.at[pl.ds(r0 + k * ZROWS, ZROWS)])
    if with_deg:
        pltpu.sync_copy(zdeg_v, deg_sh.at[pl.ds(r0, ROWS_PER_TILE)])

    plsc.subcore_barrier()

    def load_idx(i, sbuf, tbuf):
        j = pl.multiple_of((wid * NSTEPS + i) * CHUNK, CHUNK)
        pltpu.sync_copy(src_hbm.at[pl.ds(j, CHUNK)], sbuf)
        pltpu.sync_copy(dst_hbm.at[pl.ds(j, CHUNK)], tbuf)

    def scatter(rbuf, tbuf):
        pltpu.sync_copy(rbuf, acc_sh.at[tbuf], add=True)
        if with_deg:
            pltpu.sync_copy(ones_v, deg_sh.at[tbuf], add=True)

    load_idx(0, s_a, t_a)
    pltpu.async_copy(x_hbm.at[s_a], r_a, sem_g)

    def step(k, _):
        load_idx(2 * k + 1, s_b, t_b)
        pltpu.async_copy(x_hbm.at[s_b], r_b, sem_g)
        pltpu.make_async_copy(x_hbm.at[s_a], r_a, sem_g).wait()
        scatter(r_a, t_a)
        load_idx(2 * k + 2, s_a, t_a)
        pltpu.async_copy(x_hbm.at[s_a], r_a, sem_g)
        pltpu.make_async_copy(x_hbm.at[s_b], r_b, sem_g).wait()
        scatter(r_b, t_b)
        return 0
    lax.fori_loop(0, NSTEPS // 2, step, 0)
    pltpu.make_async_copy(x_hbm.at[s_a], r_a, sem_g).wait()
    plsc.subcore_barrier()

    pltpu.sync_copy(acc_sh.at[pl.ds(r0, ROWS_PER_TILE)],
                    acc_out.at[c, pl.ds(r0, ROWS_PER_TILE)])
    if with_deg:
        pltpu.sync_copy(deg_sh.at[pl.ds(r0, ROWS_PER_TILE)],
                        deg_out.at[pl.ds(c * N_PAD + r0, ROWS_PER_TILE)])


def _make_seg_sum(with_deg, d):
    srcb = [pltpu.VMEM((CHUNK,), jnp.int32) for _ in range(NBUF)]
    dstb = [pltpu.VMEM((CHUNK,), jnp.int32) for _ in range(NBUF)]
    rows = [pltpu.VMEM((CHUNK, d), jnp.float32) for _ in range(NBUF)]
    sems = [pltpu.SemaphoreType.DMA]
    if with_deg:
        out_type = (jax.ShapeDtypeStruct((NC, N_PAD, d), jnp.float32),
                    jax.ShapeDtypeStruct((NC * N_PAD,), jnp.float32))
        scratch = srcb + dstb + rows + [
            pltpu.VMEM((CHUNK,), jnp.float32),
            pltpu.VMEM((ROWS_PER_TILE,), jnp.float32),
            pltpu.VMEM_SHARED((N_PAD, d), jnp.float32),
            pltpu.VMEM_SHARED((N_PAD,), jnp.float32),
        ] + sems
    else:
        out_type = jax.ShapeDtypeStruct((NC, N_PAD, d), jnp.float32)
        scratch = srcb + dstb + rows + [
            pltpu.VMEM_SHARED((N_PAD, d), jnp.float32),
        ] + sems
    return pl.kernel(
        functools.partial(_seg_sum_body, with_deg),
        out_type=out_type,
        mesh=plsc.VectorSubcoreMesh(core_axis_name="c", subcore_axis_name="s"),
        scratch_types=scratch,
    )


_BLK = 1024
_GRID = N_PAD // _BLK


def _dot(a, b):
    return jnp.dot(a, b, preferred_element_type=jnp.float32,
                   precision=lax.Precision.HIGHEST)


def _mid_body(x_ref, a0_ref, a1_ref, d0_ref, d1_ref, ws1_ref, wn1_ref,
              b1_ref, ws2_ref, wn2_ref, b2_ref, u_ref, p2_ref):
    deg = jnp.maximum(d0_ref[...] + d1_ref[...], 1.0)
    mean = (a0_ref[...] + a1_ref[...]) / deg
    h = _dot(x_ref[...], ws1_ref[...]) + _dot(mean, wn1_ref[...]) + b1_ref[...]
    h = jnp.maximum(h, 0.0)
    u_ref[...] = _dot(h, ws2_ref[...]) + b2_ref[...]
    p2_ref[...] = _dot(h, wn2_ref[...])


def _fin_body(u_ref, a0_ref, a1_ref, d0_ref, d1_ref, out_ref):
    deg = jnp.maximum(d0_ref[...] + d1_ref[...], 1.0)
    out_ref[...] = u_ref[...] + (a0_ref[...] + a1_ref[...]) / deg


def _row_spec(d):
    return pl.BlockSpec((_BLK, d), lambda i: (i, 0))


def _full_spec(r, c):
    return pl.BlockSpec((r, c), lambda i: (0, 0))


def kernel(x, edge_index, W_self1, W_neigh1, b1, W_self2, W_neigh2, b2):
    src = edge_index[0].astype(jnp.int32)
    dst = edge_index[1].astype(jnp.int32)
    npad_e = E_ALLOC - N_EDGES
    pad_src = jnp.arange(npad_e, dtype=jnp.int32) % N_NODES
    pad_dst = N_NODES + jnp.arange(npad_e, dtype=jnp.int32) % (N_PAD - N_NODES)
    src2 = jnp.concatenate([src, pad_src])
    dst2 = jnp.concatenate([dst, pad_dst])
    x_pad = jnp.pad(x, ((0, N_PAD - N_NODES), (0, 0)))

    seg1 = _make_seg_sum(True, D_IN)
    agg1, deg_flat = seg1(x_pad, src2, dst2)
    deg2 = deg_flat.reshape(NC, N_PAD)
    d0 = deg2[0][:, None]
    d1 = deg2[1][:, None]

    mid = pl.pallas_call(
        _mid_body,
        grid=(_GRID,),
        in_specs=[
            _row_spec(D_IN), _row_spec(D_IN), _row_spec(D_IN),
            _row_spec(1), _row_spec(1),
            _full_spec(D_IN, D_HID), _full_spec(D_IN, D_HID),
            _full_spec(1, D_HID),
            _full_spec(D_HID, D_OUT), _full_spec(D_HID, D_OUT),
            _full_spec(1, D_OUT),
        ],
        out_specs=[_row_spec(D_OUT), _row_spec(D_OUT)],
        out_shape=[
            jax.ShapeDtypeStruct((N_PAD, D_OUT), jnp.float32),
            jax.ShapeDtypeStruct((N_PAD, D_OUT), jnp.float32),
        ],
    )
    u, p2 = mid(x_pad, agg1[0], agg1[1], d0, d1,
                W_self1, W_neigh1, b1.reshape(1, D_HID),
                W_self2, W_neigh2, b2.reshape(1, D_OUT))

    seg2 = _make_seg_sum(False, D_OUT)
    agg2 = seg2(p2, src2, dst2)

    fin = pl.pallas_call(
        _fin_body,
        grid=(_GRID,),
        in_specs=[
            _row_spec(D_OUT), _row_spec(D_OUT), _row_spec(D_OUT),
            _row_spec(1), _row_spec(1),
        ],
        out_specs=_row_spec(D_OUT),
        out_shape=jax.ShapeDtypeStruct((N_PAD, D_OUT), jnp.float32),
    )
    out = fin(u, agg2[0], agg2[1], d0, d1)
    return out[:N_NODES]


# speedup vs baseline: 3.6571x; 1.0932x over previous
import functools

import jax
import jax.numpy as jnp
from jax import lax
from jax.experimental import pallas as pl
from jax.experimental.pallas import tpu as pltpu
from jax.experimental.pallas import tpu_sc as plsc

N_NODES = 10000
N_EDGES = 320000
D_IN = 128
D_HID = 256
D_OUT = 128

NC = 2
NS = 16
NW = NC * NS
CHUNK = 128
NSTEPS = 80
E_PAD = NW * NSTEPS * CHUNK
NBUF = 2
NIDX = 4
E_ALLOC = E_PAD + NIDX * CHUNK
N_PAD = 10240
ROWS_PER_TILE = N_PAD // NS
ZROWS = CHUNK


def _seg_sum_body(with_deg, x_hbm, src_hbm, dst_hbm, *refs):
    if with_deg:
        (acc_out, deg_out, s_0, s_1, s_2, s_3, t_0, t_1, t_2, t_3, r_a, r_b,
         ones_v, zdeg_v, acc_sh, deg_sh, sem_ga, sem_gb,
         sem_i0, sem_i1, sem_i2, sem_i3) = refs
    else:
        (acc_out, s_0, s_1, s_2, s_3, t_0, t_1, t_2, t_3, r_a, r_b,
         acc_sh, sem_ga, sem_gb, sem_i0, sem_i1, sem_i2, sem_i3) = refs
        deg_out = deg_sh = ones_v = zdeg_v = None
    sp = [s_0, s_1, s_2, s_3]
    tp = [t_0, t_1, t_2, t_3]
    sem_i = [sem_i0, sem_i1, sem_i2, sem_i3]
    sem_r = {id(r_a): sem_ga, id(r_b): sem_gb}

    c = lax.axis_index("c")
    s = lax.axis_index("s")
    wid = s * NC + c

    def zb(i, _):
        r = i // 8
        col = (i % 8) * 16
        r_a[r, pl.ds(col, 16)] = jnp.zeros((16,), jnp.float32)
        return 0
    lax.fori_loop(0, ZROWS * 8, zb, 0)
    if with_deg:
        def ob(i, _):
            ones_v[pl.ds(i * 16, 16)] = jnp.ones((16,), jnp.float32)
            return 0
        lax.fori_loop(0, CHUNK // 16, ob, 0)

        def zd(i, _):
            zdeg_v[pl.ds(i * 16, 16)] = jnp.zeros((16,), jnp.float32)
            return 0
        lax.fori_loop(0, ROWS_PER_TILE // 16, zd, 0)

    r0 = s * ROWS_PER_TILE
    for k in range(ROWS_PER_TILE // ZROWS):
        pltpu.sync_copy(r_a, acc_sh.at[pl.ds(r0 + k * ZROWS, ZROWS)])
    if with_deg:
        pltpu.sync_copy(zdeg_v, deg_sh.at[pl.ds(r0, ROWS_PER_TILE)])

    plsc.subcore_barrier()

    def idx_start(i, p):
        j = pl.multiple_of((wid * NSTEPS + i) * CHUNK, CHUNK)
        pltpu.async_copy(src_hbm.at[pl.ds(j, CHUNK)], sp[p], sem_i[p])
        pltpu.async_copy(dst_hbm.at[pl.ds(j, CHUNK)], tp[p], sem_i[p])

    def idx_wait(p):
        pltpu.make_async_copy(src_hbm.at[pl.ds(0, CHUNK)], sp[p], sem_i[p]).wait()
        pltpu.make_async_copy(dst_hbm.at[pl.ds(0, CHUNK)], tp[p], sem_i[p]).wait()

    def gather_start(p, rbuf):
        pltpu.async_copy(x_hbm.at[sp[p]], rbuf, sem_r[id(rbuf)])

    def gather_wait(p, rbuf):
        pltpu.make_async_copy(x_hbm.at[sp[p]], rbuf, sem_r[id(rbuf)]).wait()

    def scatter(rbuf, p):
        pltpu.sync_copy(rbuf, acc_sh.at[tp[p]], add=True)
        if with_deg:
            pltpu.sync_copy(ones_v, deg_sh.at[tp[p]], add=True)

    for p in range(NIDX):
        idx_start(p, p)
    idx_wait(0)
    gather_start(0, r_a)

    def step(q, _):
        j4 = 4 * q + 4
        idx_wait(1)
        gather_start(1, r_b)
        gather_wait(0, r_a)
        scatter(r_a, 0)
        idx_start(j4, 0)
        idx_wait(2)
        gather_start(2, r_a)
        gather_wait(1, r_b)
        scatter(r_b, 1)
        idx_start(j4 + 1, 1)
        idx_wait(3)
        gather_start(3, r_b)
        gather_wait(2, r_a)
        scatter(r_a, 2)
        idx_start(j4 + 2, 2)
        gather_wait(3, r_b)
        scatter(r_b, 3)
        idx_start(j4 + 3, 3)
        idx_wait(0)
        gather_start(0, r_a)
        return 0
    lax.fori_loop(0, NSTEPS // 4, step, 0)
    gather_wait(0, r_a)
    for p in range(1, NIDX):
        idx_wait(p)
    plsc.subcore_barrier()

    pltpu.sync_copy(acc_sh.at[pl.ds(r0, ROWS_PER_TILE)],
                    acc_out.at[c, pl.ds(r0, ROWS_PER_TILE)])
    if with_deg:
        pltpu.sync_copy(deg_sh.at[pl.ds(r0, ROWS_PER_TILE)],
                        deg_out.at[pl.ds(c * N_PAD + r0, ROWS_PER_TILE)])


def _make_seg_sum(with_deg, d):
    srcb = [pltpu.VMEM((CHUNK,), jnp.int32) for _ in range(NIDX)]
    dstb = [pltpu.VMEM((CHUNK,), jnp.int32) for _ in range(NIDX)]
    rows = [pltpu.VMEM((CHUNK, d), jnp.float32) for _ in range(NBUF)]
    sems = [pltpu.SemaphoreType.DMA for _ in range(2 + NIDX)]
    if with_deg:
        out_type = (jax.ShapeDtypeStruct((NC, N_PAD, d), jnp.float32),
                    jax.ShapeDtypeStruct((NC * N_PAD,), jnp.float32))
        scratch = srcb + dstb + rows + [
            pltpu.VMEM((CHUNK,), jnp.float32),
            pltpu.VMEM((ROWS_PER_TILE,), jnp.float32),
            pltpu.VMEM_SHARED((N_PAD, d), jnp.float32),
            pltpu.VMEM_SHARED((N_PAD,), jnp.float32),
        ] + sems
    else:
        out_type = jax.ShapeDtypeStruct((NC, N_PAD, d), jnp.float32)
        scratch = srcb + dstb + rows + [
            pltpu.VMEM_SHARED((N_PAD, d), jnp.float32),
        ] + sems
    return pl.kernel(
        functools.partial(_seg_sum_body, with_deg),
        out_type=out_type,
        mesh=plsc.VectorSubcoreMesh(core_axis_name="c", subcore_axis_name="s"),
        scratch_types=scratch,
    )


_BLK = 1024
_GRID = N_PAD // _BLK


def _dot(a, b):
    return jnp.dot(a, b, preferred_element_type=jnp.float32,
                   precision=lax.Precision.HIGHEST)


def _mid_body(x_ref, a0_ref, a1_ref, d0_ref, d1_ref, ws1_ref, wn1_ref,
              b1_ref, ws2_ref, wn2_ref, b2_ref, u_ref, p2_ref):
    deg = jnp.maximum(d0_ref[...] + d1_ref[...], 1.0)
    mean = (a0_ref[...] + a1_ref[...]) / deg
    h = _dot(x_ref[...], ws1_ref[...]) + _dot(mean, wn1_ref[...]) + b1_ref[...]
    h = jnp.maximum(h, 0.0)
    u_ref[...] = _dot(h, ws2_ref[...]) + b2_ref[...]
    p2_ref[...] = _dot(h, wn2_ref[...])


def _fin_body(u_ref, a0_ref, a1_ref, d0_ref, d1_ref, out_ref):
    deg = jnp.maximum(d0_ref[...] + d1_ref[...], 1.0)
    out_ref[...] = u_ref[...] + (a0_ref[...] + a1_ref[...]) / deg


def _row_spec(d):
    return pl.BlockSpec((_BLK, d), lambda i: (i, 0))


def _full_spec(r, c):
    return pl.BlockSpec((r, c), lambda i: (0, 0))


def kernel(x, edge_index, W_self1, W_neigh1, b1, W_self2, W_neigh2, b2):
    src = edge_index[0].astype(jnp.int32)
    dst = edge_index[1].astype(jnp.int32)
    npad_e = E_ALLOC - N_EDGES
    pad_src = jnp.arange(npad_e, dtype=jnp.int32) % N_NODES
    pad_dst = N_NODES + jnp.arange(npad_e, dtype=jnp.int32) % (N_PAD - N_NODES)
    src2 = jnp.concatenate([src, pad_src])
    dst2 = jnp.concatenate([dst, pad_dst])
    x_pad = jnp.pad(x, ((0, N_PAD - N_NODES), (0, 0)))

    seg1 = _make_seg_sum(True, D_IN)
    agg1, deg_flat = seg1(x_pad, src2, dst2)
    deg2 = deg_flat.reshape(NC, N_PAD)
    d0 = deg2[0][:, None]
    d1 = deg2[1][:, None]

    mid = pl.pallas_call(
        _mid_body,
        grid=(_GRID,),
        in_specs=[
            _row_spec(D_IN), _row_spec(D_IN), _row_spec(D_IN),
            _row_spec(1), _row_spec(1),
            _full_spec(D_IN, D_HID), _full_spec(D_IN, D_HID),
            _full_spec(1, D_HID),
            _full_spec(D_HID, D_OUT), _full_spec(D_HID, D_OUT),
            _full_spec(1, D_OUT),
        ],
        out_specs=[_row_spec(D_OUT), _row_spec(D_OUT)],
        out_shape=[
            jax.ShapeDtypeStruct((N_PAD, D_OUT), jnp.float32),
            jax.ShapeDtypeStruct((N_PAD, D_OUT), jnp.float32),
        ],
    )
    u, p2 = mid(x_pad, agg1[0], agg1[1], d0, d1,
                W_self1, W_neigh1, b1.reshape(1, D_HID),
                W_self2, W_neigh2, b2.reshape(1, D_OUT))

    seg2 = _make_seg_sum(False, D_OUT)
    agg2 = seg2(p2, src2, dst2)

    fin = pl.pallas_call(
        _fin_body,
        grid=(_GRID,),
        in_specs=[
            _row_spec(D_OUT), _row_spec(D_OUT), _row_spec(D_OUT),
            _row_spec(1), _row_spec(1),
        ],
        out_specs=_row_spec(D_OUT),
        out_shape=jax.ShapeDtypeStruct((N_PAD, D_OUT), jnp.float32),
    )
    out = fin(u, agg2[0], agg2[1], d0, d1)
    return out[:N_NODES]


# speedup vs baseline: 3.7289x; 1.0196x over previous
import functools

import jax
import jax.numpy as jnp
from jax import lax
from jax.experimental import pallas as pl
from jax.experimental.pallas import tpu as pltpu
from jax.experimental.pallas import tpu_sc as plsc

N_NODES = 10000
N_EDGES = 320000
D_IN = 128
D_HID = 256
D_OUT = 128

NC = 2
NS = 16
NW = NC * NS
CHUNK = 128
NSTEPS = 80
E_PAD = NW * NSTEPS * CHUNK
NBUF = 2
NIDX = 4
E_ALLOC = E_PAD + NIDX * CHUNK
N_PAD = 10240
ROWS_PER_TILE = N_PAD // NS
ZROWS = CHUNK


def _seg_sum_body(with_deg, x_hbm, src_hbm, dst_hbm, *refs):
    if with_deg:
        (acc_out, deg_out, s_0, s_1, s_2, s_3, t_0, t_1, t_2, t_3, r_a, r_b,
         ones_v, zdeg_v, acc_sh, deg_sh, sem_ga, sem_gb,
         sem_i0, sem_i1, sem_i2, sem_i3) = refs
    else:
        (acc_out, s_0, s_1, s_2, s_3, t_0, t_1, t_2, t_3, r_a, r_b,
         acc_sh, sem_ga, sem_gb, sem_i0, sem_i1, sem_i2, sem_i3) = refs
        deg_out = deg_sh = ones_v = zdeg_v = None
    sp = [s_0, s_1, s_2, s_3]
    tp = [t_0, t_1, t_2, t_3]
    sem_i = [sem_i0, sem_i1, sem_i2, sem_i3]
    sem_r = {id(r_a): sem_ga, id(r_b): sem_gb}

    c = lax.axis_index("c")
    s = lax.axis_index("s")
    wid = s * NC + c

    def zb(i, _):
        r = i // 8
        col = (i % 8) * 16
        r_a[r, pl.ds(col, 16)] = jnp.zeros((16,), jnp.float32)
        return 0
    lax.fori_loop(0, ZROWS * 8, zb, 0)
    if with_deg:
        def ob(i, _):
            ones_v[pl.ds(i * 16, 16)] = jnp.ones((16,), jnp.float32)
            return 0
        lax.fori_loop(0, CHUNK // 16, ob, 0)

        def zd(i, _):
            zdeg_v[pl.ds(i * 16, 16)] = jnp.zeros((16,), jnp.float32)
            return 0
        lax.fori_loop(0, ROWS_PER_TILE // 16, zd, 0)

    r0 = s * ROWS_PER_TILE
    for k in range(ROWS_PER_TILE // ZROWS):
        pltpu.sync_copy(r_a, acc_sh.at[pl.ds(r0 + k * ZROWS, ZROWS)])
    if with_deg:
        pltpu.sync_copy(zdeg_v, deg_sh.at[pl.ds(r0, ROWS_PER_TILE)])

    plsc.subcore_barrier()

    def idx_start(i, p):
        j = pl.multiple_of((wid * NSTEPS + i) * CHUNK, CHUNK)
        pltpu.async_copy(src_hbm.at[pl.ds(j, CHUNK)], sp[p], sem_i[p])
        pltpu.async_copy(dst_hbm.at[pl.ds(j, CHUNK)], tp[p], sem_i[p])

    def idx_wait(p):
        pltpu.make_async_copy(src_hbm.at[pl.ds(0, CHUNK)], sp[p], sem_i[p]).wait()
        pltpu.make_async_copy(dst_hbm.at[pl.ds(0, CHUNK)], tp[p], sem_i[p]).wait()

    def gather_start(p, rbuf):
        pltpu.async_copy(x_hbm.at[sp[p]], rbuf, sem_r[id(rbuf)])

    def gather_wait(p, rbuf):
        pltpu.make_async_copy(x_hbm.at[sp[p]], rbuf, sem_r[id(rbuf)]).wait()

    def scatter(rbuf, p):
        pltpu.sync_copy(rbuf, acc_sh.at[tp[p]], add=True)
        if with_deg:
            pltpu.sync_copy(ones_v, deg_sh.at[tp[p]], add=True)

    for p in range(NIDX):
        idx_start(p, p)
    idx_wait(0)
    gather_start(0, r_a)

    def step(q, _):
        j4 = 4 * q + 4
        idx_wait(1)
        gather_start(1, r_b)
        gather_wait(0, r_a)
        scatter(r_a, 0)
        idx_start(j4, 0)
        idx_wait(2)
        gather_start(2, r_a)
        gather_wait(1, r_b)
        scatter(r_b, 1)
        idx_start(j4 + 1, 1)
        idx_wait(3)
        gather_start(3, r_b)
        gather_wait(2, r_a)
        scatter(r_a, 2)
        idx_start(j4 + 2, 2)
        gather_wait(3, r_b)
        scatter(r_b, 3)
        idx_start(j4 + 3, 3)
        idx_wait(0)
        gather_start(0, r_a)
        return 0
    lax.fori_loop(0, NSTEPS // 4, step, 0)
    gather_wait(0, r_a)
    for p in range(1, NIDX):
        idx_wait(p)
    plsc.subcore_barrier()

    pltpu.sync_copy(acc_sh.at[pl.ds(r0, ROWS_PER_TILE)],
                    acc_out.at[c, pl.ds(r0, ROWS_PER_TILE)])
    if with_deg:
        pltpu.sync_copy(deg_sh.at[pl.ds(r0, ROWS_PER_TILE)],
                        deg_out.at[pl.ds(c * N_PAD + r0, ROWS_PER_TILE)])


def _make_seg_sum(with_deg, d):
    srcb = [pltpu.VMEM((CHUNK,), jnp.int32) for _ in range(NIDX)]
    dstb = [pltpu.VMEM((CHUNK,), jnp.int32) for _ in range(NIDX)]
    rows = [pltpu.VMEM((CHUNK, d), jnp.float32) for _ in range(NBUF)]
    sems = [pltpu.SemaphoreType.DMA for _ in range(2 + NIDX)]
    if with_deg:
        out_type = (jax.ShapeDtypeStruct((NC, N_PAD, d), jnp.float32),
                    jax.ShapeDtypeStruct((NC * N_PAD,), jnp.float32))
        scratch = srcb + dstb + rows + [
            pltpu.VMEM((CHUNK,), jnp.float32),
            pltpu.VMEM((ROWS_PER_TILE,), jnp.float32),
            pltpu.VMEM_SHARED((N_PAD, d), jnp.float32),
            pltpu.VMEM_SHARED((N_PAD,), jnp.float32),
        ] + sems
    else:
        out_type = jax.ShapeDtypeStruct((NC, N_PAD, d), jnp.float32)
        scratch = srcb + dstb + rows + [
            pltpu.VMEM_SHARED((N_PAD, d), jnp.float32),
        ] + sems
    return pl.kernel(
        functools.partial(_seg_sum_body, with_deg),
        out_type=out_type,
        mesh=plsc.VectorSubcoreMesh(core_axis_name="c", subcore_axis_name="s"),
        scratch_types=scratch,
    )


_BLK = 1000
_GRID = N_NODES // _BLK


def _dot(a, b):
    return jnp.dot(a, b, preferred_element_type=jnp.float32,
                   precision=lax.Precision.HIGHEST)


def _mid_body(x_ref, a0_ref, a1_ref, d0_ref, d1_ref, ws1_ref, wn1_ref,
              b1_ref, ws2_ref, wn2_ref, b2_ref, u_ref, p2_ref):
    deg = jnp.maximum(d0_ref[...] + d1_ref[...], 1.0)
    mean = (a0_ref[...] + a1_ref[...]) / deg
    h = _dot(x_ref[...], ws1_ref[...]) + _dot(mean, wn1_ref[...]) + b1_ref[...]
    h = jnp.maximum(h, 0.0)
    u_ref[...] = _dot(h, ws2_ref[...]) + b2_ref[...]
    p2_ref[...] = _dot(h, wn2_ref[...])


def _fin_body(u_ref, a0_ref, a1_ref, d0_ref, d1_ref, out_ref):
    deg = jnp.maximum(d0_ref[...] + d1_ref[...], 1.0)
    out_ref[...] = u_ref[...] + (a0_ref[...] + a1_ref[...]) / deg


def _row_spec(d):
    return pl.BlockSpec((_BLK, d), lambda i: (i, 0))


def _full_spec(r, c):
    return pl.BlockSpec((r, c), lambda i: (0, 0))


def kernel(x, edge_index, W_self1, W_neigh1, b1, W_self2, W_neigh2, b2):
    src = edge_index[0].astype(jnp.int32)
    dst = edge_index[1].astype(jnp.int32)
    npad_e = E_ALLOC - N_EDGES
    pad_src = jnp.arange(npad_e, dtype=jnp.int32) % N_NODES
    pad_dst = N_NODES + jnp.arange(npad_e, dtype=jnp.int32) % (N_PAD - N_NODES)
    src2 = jnp.concatenate([src, pad_src])
    dst2 = jnp.concatenate([dst, pad_dst])
    seg1 = _make_seg_sum(True, D_IN)
    agg1, deg_flat = seg1(x, src2, dst2)
    deg2 = deg_flat.reshape(NC, N_PAD)
    d0 = deg2[0][:, None]
    d1 = deg2[1][:, None]

    mid = pl.pallas_call(
        _mid_body,
        grid=(_GRID,),
        in_specs=[
            _row_spec(D_IN), _row_spec(D_IN), _row_spec(D_IN),
            _row_spec(1), _row_spec(1),
            _full_spec(D_IN, D_HID), _full_spec(D_IN, D_HID),
            _full_spec(1, D_HID),
            _full_spec(D_HID, D_OUT), _full_spec(D_HID, D_OUT),
            _full_spec(1, D_OUT),
        ],
        out_specs=[_row_spec(D_OUT), _row_spec(D_OUT)],
        out_shape=[
            jax.ShapeDtypeStruct((N_NODES, D_OUT), jnp.float32),
            jax.ShapeDtypeStruct((N_NODES, D_OUT), jnp.float32),
        ],
    )
    u, p2 = mid(x, agg1[0], agg1[1], d0, d1,
                W_self1, W_neigh1, b1.reshape(1, D_HID),
                W_self2, W_neigh2, b2.reshape(1, D_OUT))

    seg2 = _make_seg_sum(False, D_OUT)
    agg2 = seg2(p2, src2, dst2)

    fin = pl.pallas_call(
        _fin_body,
        grid=(_GRID,),
        in_specs=[
            _row_spec(D_OUT), _row_spec(D_OUT), _row_spec(D_OUT),
            _row_spec(1), _row_spec(1),
        ],
        out_specs=_row_spec(D_OUT),
        out_shape=jax.ShapeDtypeStruct((N_NODES, D_OUT), jnp.float32),
    )
    return fin(u, agg2[0], agg2[1], d0, d1)
